# Initial kernel scaffold; baseline (speedup 1.0000x reference)
import functools

import jax
import jax.numpy as jnp
from jax import lax
from jax.experimental import pallas as pl
from jax.experimental.pallas import tpu as pltpu
from jax.experimental.pallas import tpu_sc as plsc

_N = 10000
_E = 320000
_D = 128

_NC = 2
_NS = 16
_NW = _NC * _NS
_EPW = _E // _NW
_B = 80
_NB = _EPW // _B
_G = _B // 16
_RPW = _N // _NS
_ZR = 25


def _pre_body(x_ref, w_ref, asrc_ref, adst_ref, h_ref, as_ref, ad_ref):
    hv = jnp.dot(x_ref[...], w_ref[...].T,
                 precision=lax.Precision.HIGHEST,
                 preferred_element_type=jnp.float32)
    h_ref[...] = hv
    as_ref[...] = jnp.sum(hv * asrc_ref[...][None, :], axis=1)
    ad_ref[...] = jnp.sum(hv * adst_ref[...][None, :], axis=1)


def _sc_body(h_hbm, edge_hbm, as_hbm, ad_hbm, acc_hbm, den_hbm,
             as_v, ad_v, src_v, dst_v, rows_v, stage_v, zbuf, zden,
             acc_sh, den_sh, sem):
    cid = lax.axis_index("c")
    sid = lax.axis_index("s")
    w = cid * _NS + sid

    pltpu.sync_copy(as_hbm, as_v)
    pltpu.sync_copy(ad_hbm, ad_v)

    z16 = jnp.zeros((16,), jnp.float32)
    for r in range(_ZR):
        zden[r, :] = z16
        for c in range(_D // 16):
            zbuf[r, pl.ds(c * 16, 16)] = z16
    for r in range(_B):
        stage_v[r, :] = z16

    row0 = sid * _RPW

    @pl.loop(0, _RPW // _ZR)
    def _zero(k):
        pltpu.sync_copy(zbuf, acc_sh.at[pl.ds(row0 + k * _ZR, _ZR)])
        pltpu.sync_copy(zden, den_sh.at[pl.ds(row0 + k * _ZR, _ZR)])

    plsc.subcore_barrier()

    iota16 = lax.iota(jnp.int32, 16)
    zi16 = jnp.zeros((16,), jnp.int32)
    ebase = w * _EPW

    @pl.loop(0, _NB)
    def _batch(i):
        eb = ebase + i * _B
        pltpu.sync_copy(edge_hbm.at[0, pl.ds(eb, _B)], src_v)
        pltpu.sync_copy(edge_hbm.at[1, pl.ds(eb, _B)], dst_v)
        gat = pltpu.async_copy(h_hbm.at[src_v], rows_v, sem)

        for g in range(_G):
            sg = src_v[pl.ds(g * 16, 16)]
            dg = dst_v[pl.ds(g * 16, 16)]
            e = plsc.load_gather(as_v, [sg]) + plsc.load_gather(ad_v, [dg])
            e = jnp.where(e > 0.0, e, 0.2 * e)
            plsc.store_scatter(stage_v, [g * 16 + iota16, zi16], jnp.exp(e))

        gat.wait()

        for e_ in range(_B):
            splat = plsc.load_gather(
                stage_v, [jnp.full((16,), e_, jnp.int32), zi16])
            for c in range(_D // 16):
                sl = pl.ds(c * 16, 16)
                rows_v[e_, sl] = rows_v[e_, sl] * splat

        pltpu.sync_copy(rows_v, acc_sh.at[dst_v], add=True)
        pltpu.sync_copy(stage_v, den_sh.at[dst_v], add=True)

    plsc.subcore_barrier()

    pltpu.sync_copy(acc_sh.at[pl.ds(row0, _RPW)],
                    acc_hbm.at[cid, pl.ds(row0, _RPW)])
    pltpu.sync_copy(den_sh.at[pl.ds(row0, _RPW)],
                    den_hbm.at[cid, pl.ds(row0, _RPW)])


def _post_body(acc_ref, den_ref, x_ref, bias_ref, gamma_ref, beta_ref, o_ref):
    out = acc_ref[0] + acc_ref[1]
    den = den_ref[0, :, 0] + den_ref[1, :, 0]
    out = out / (den + 1e-16)[:, None] + bias_ref[...][None, :]
    mean = jnp.mean(out, axis=0)
    cent = out - mean[None, :]
    var = jnp.mean(cent * cent, axis=0)
    y = cent * lax.rsqrt(var + 1e-5) * gamma_ref[...][None, :] \
        + beta_ref[...][None, :]
    o_ref[...] = x_ref[...] + jnp.maximum(y, 0.0)


def kernel(x, edge_index, W, att_src, att_dst, bias, gamma, beta):
    h, a_s, a_d = pl.pallas_call(
        _pre_body,
        out_shape=[
            jax.ShapeDtypeStruct((_N, _D), jnp.float32),
            jax.ShapeDtypeStruct((_N,), jnp.float32),
            jax.ShapeDtypeStruct((_N,), jnp.float32),
        ],
    )(x, W, att_src, att_dst)

    sc_fn = pl.kernel(
        _sc_body,
        out_type=[
            jax.ShapeDtypeStruct((_NC, _N, _D), jnp.float32),
            jax.ShapeDtypeStruct((_NC, _N, 16), jnp.float32),
        ],
        mesh=plsc.VectorSubcoreMesh(core_axis_name="c", subcore_axis_name="s"),
        scratch_types=[
            pltpu.VMEM((_N,), jnp.float32),
            pltpu.VMEM((_N,), jnp.float32),
            pltpu.VMEM((_B,), jnp.int32),
            pltpu.VMEM((_B,), jnp.int32),
            pltpu.VMEM((_B, _D), jnp.float32),
            pltpu.VMEM((_B, 16), jnp.float32),
            pltpu.VMEM((_ZR, _D), jnp.float32),
            pltpu.VMEM((_ZR, 16), jnp.float32),
            pltpu.VMEM_SHARED((_N, _D), jnp.float32),
            pltpu.VMEM_SHARED((_N, 16), jnp.float32),
            pltpu.SemaphoreType.DMA,
        ],
    )
    acc, den = sc_fn(h, edge_index, a_s, a_d)

    return pl.pallas_call(
        _post_body,
        out_shape=jax.ShapeDtypeStruct((_N, _D), jnp.float32),
    )(acc, den, x, bias, gamma, beta)


# speedup vs baseline: 12.4062x; 12.4062x over previous
import dataclasses
import functools

import jax
import jax.numpy as jnp
from jax import lax
from jax.experimental import pallas as pl
from jax.experimental.pallas import tpu as pltpu
from jax.experimental.pallas import tpu_sc as plsc

_N = 10000
_E = 320000
_D = 128

_DIAG_XLA_EDGE_PHASE = False

_NC = 2
_NS = 16
_DH = _D // _NC
_EPW = _E // _NS
_B = 80
_NB = _EPW // _B
_G = _B // 16
_NP = 10240
_RPW = _NP // _NS
_ZR = 32


def _sc_compiler_params():
    cp = pltpu.CompilerParams()
    fields = pltpu.CompilerParams.__dataclass_fields__
    if "needs_layout_passes" in fields:
        cp = dataclasses.replace(cp, needs_layout_passes=False)
    if "use_tc_tiling_on_sc" in fields:
        cp = dataclasses.replace(cp, use_tc_tiling_on_sc=False)
    return cp


def _pre_body(x_ref, w_ref, asrc_ref, adst_ref, h_ref, as_ref, ad_ref):
    hv = jnp.dot(x_ref[...], w_ref[...].T,
                 precision=lax.Precision.HIGHEST,
                 preferred_element_type=jnp.float32)
    h_ref[...] = hv
    as_ref[...] = jnp.sum(hv * asrc_ref[...][None, :], axis=1)
    ad_ref[...] = jnp.sum(hv * adst_ref[...][None, :], axis=1)


def _sc_body(h_hbm, edge_hbm, as_hbm, ad_hbm, acc_hbm, den_hbm,
             as_v, ad_v, src_v, dst_v, src2_v, ex_v, rows_v, stage_v,
             zbuf, zden, acc_sh, den_sh, sem):
    cid = lax.axis_index("c")
    sid = lax.axis_index("s")

    pltpu.sync_copy(as_hbm, as_v)
    pltpu.sync_copy(ad_hbm, ad_v)

    z16 = jnp.zeros((16,), jnp.float32)
    for r in range(_ZR):
        zden[r, :] = z16
        for c in range(_DH // 16):
            zbuf[r, pl.ds(c * 16, 16)] = z16
    for r in range(_B):
        stage_v[r, :] = z16

    row0 = sid * _RPW

    @pl.loop(0, _RPW // _ZR)
    def _zero(k):
        pltpu.sync_copy(zbuf, acc_sh.at[pl.ds(row0 + k * _ZR, _ZR)])

        @pl.when(cid == 0)
        def _():
            pltpu.sync_copy(zden, den_sh.at[pl.ds(row0 + k * _ZR, _ZR)])
    del _zero

    plsc.subcore_barrier()

    iota16 = lax.iota(jnp.int32, 16)
    zi16 = jnp.zeros((16,), jnp.int32)

    def _one_batch(eb):
        pltpu.sync_copy(edge_hbm.at[pl.ds(eb, _B)], src_v)
        pltpu.sync_copy(edge_hbm.at[pl.ds(_E + eb, _B)], dst_v)

        for g in range(_G):
            sl = pl.ds(g * 16, 16)
            src2_v[sl] = src_v[sl] * 2 + cid
        gat = pltpu.async_copy(h_hbm.at[src2_v], rows_v, sem)

        for g in range(_G):
            sg = src_v[pl.ds(g * 16, 16)]
            dg = dst_v[pl.ds(g * 16, 16)]
            e = plsc.load_gather(as_v, [sg]) + plsc.load_gather(ad_v, [dg])
            e = jnp.where(e > 0.0, e, 0.2 * e)
            ex16 = jnp.exp(e)
            ex_v[pl.ds(16 + g * 16, 16)] = ex16
            plsc.store_scatter(stage_v, [g * 16 + iota16, zi16], ex16)

        gat.wait()

        for e_ in range(_B):
            splat = plsc.load_gather(
                ex_v, [jnp.full((16,), 16 + e_, jnp.int32)])
            for c in range(_DH // 16):
                sl = pl.ds(c * 16, 16)
                rows_v[e_, sl] = rows_v[e_, sl] * splat

        pltpu.sync_copy(rows_v, acc_sh.at[dst_v], add=True)

        @pl.when(cid == 0)
        def _():
            pltpu.sync_copy(stage_v, den_sh.at[dst_v], add=True)

    ebase = sid * _EPW

    @pl.loop(0, _NB)
    def _batch(i):
        _one_batch(ebase + i * _B)

    plsc.subcore_barrier()

    pltpu.sync_copy(acc_sh.at[pl.ds(row0, _RPW)],
                    acc_hbm.at[cid, pl.ds(row0, _RPW)])

    @pl.when(cid == 0)
    def _copy_den():
        pltpu.sync_copy(den_sh.at[pl.ds(row0, _RPW)],
                        den_hbm.at[pl.ds(row0, _RPW)])
    del _copy_den


def _post_body(acc_ref, den_ref, x_ref, bias_ref, gamma_ref, beta_ref, o_ref):
    out = jnp.concatenate([acc_ref[0, :_N], acc_ref[1, :_N]], axis=1)
    den = den_ref[:_N, 0]
    out = out / (den + 1e-16)[:, None] + bias_ref[...][None, :]
    mean = jnp.mean(out, axis=0)
    cent = out - mean[None, :]
    var = jnp.mean(cent * cent, axis=0)
    y = cent * lax.rsqrt(var + 1e-5) * gamma_ref[...][None, :] \
        + beta_ref[...][None, :]
    o_ref[...] = x_ref[...] + jnp.maximum(y, 0.0)


def kernel(x, edge_index, W, att_src, att_dst, bias, gamma, beta):
    h, a_s, a_d = pl.pallas_call(
        _pre_body,
        out_shape=[
            jax.ShapeDtypeStruct((_N, _D), jnp.float32),
            jax.ShapeDtypeStruct((_N,), jnp.float32),
            jax.ShapeDtypeStruct((_N,), jnp.float32),
        ],
    )(x, W, att_src, att_dst)

    sc_fn = pl.kernel(
        _sc_body,
        out_type=[
            jax.ShapeDtypeStruct((_NC, _NP, _DH), jnp.float32),
            jax.ShapeDtypeStruct((_NP, 16), jnp.float32),
        ],
        mesh=plsc.VectorSubcoreMesh(core_axis_name="c", subcore_axis_name="s"),
        compiler_params=_sc_compiler_params(),
        scratch_types=[
            pltpu.VMEM((_N,), jnp.float32),
            pltpu.VMEM((_N,), jnp.float32),
            pltpu.VMEM((_B,), jnp.int32),
            pltpu.VMEM((_B,), jnp.int32),
            pltpu.VMEM((_B,), jnp.int32),
            pltpu.VMEM((16 + _B,), jnp.float32),
            pltpu.VMEM((_B, _DH), jnp.float32),
            pltpu.VMEM((_B, 16), jnp.float32),
            pltpu.VMEM((_ZR, _DH), jnp.float32),
            pltpu.VMEM((_ZR, 16), jnp.float32),
            pltpu.VMEM_SHARED((_NP, _DH), jnp.float32),
            pltpu.VMEM_SHARED((_NP, 16), jnp.float32),
            pltpu.SemaphoreType.DMA,
        ],
    )
    if _DIAG_XLA_EDGE_PHASE:
        src = edge_index[0]
        dst = edge_index[1]
        e = a_s[src] + a_d[dst]
        e = jnp.where(e > 0, e, 0.2 * e)
        ex = jnp.exp(e)
        den_full = jax.ops.segment_sum(ex, dst, num_segments=_NP)
        accf = jax.ops.segment_sum(h[src] * ex[:, None], dst,
                                   num_segments=_NP)
        acc = jnp.stack([accf[:, :_DH], accf[:, _DH:]])
        den = jnp.zeros((_NP, 16), jnp.float32).at[:, 0].set(den_full)
    else:
        acc, den = sc_fn(h.reshape(2 * _N, _DH), edge_index.reshape(2 * _E),
                         a_s, a_d)

    return pl.pallas_call(
        _post_body,
        out_shape=jax.ShapeDtypeStruct((_N, _D), jnp.float32),
    )(acc, den, x, bias, gamma, beta)


# speedup vs baseline: 13.7626x; 1.1093x over previous
import dataclasses
import functools

import jax
import jax.numpy as jnp
from jax import lax
from jax.experimental import pallas as pl
from jax.experimental.pallas import tpu as pltpu
from jax.experimental.pallas import tpu_sc as plsc

_N = 10000
_E = 320000
_D = 128

_NC = 2
_NS = 16
_DH = _D // _NC
_B = 80
_NBT = _E // _B
_NB = _NBT // _NS
_G = _B // 16
_NP = 10240
_RPW = _NP // _NS
_ZR = 32


def _sc_compiler_params():
    cp = pltpu.CompilerParams()
    fields = pltpu.CompilerParams.__dataclass_fields__
    if "needs_layout_passes" in fields:
        cp = dataclasses.replace(cp, needs_layout_passes=False)
    if "use_tc_tiling_on_sc" in fields:
        cp = dataclasses.replace(cp, use_tc_tiling_on_sc=False)
    return cp


def _pre_body(x_ref, w_ref, att_ref, h_ref, asd_ref):
    xv = x_ref[...]
    wv = w_ref[...]
    h_ref[...] = jnp.dot(xv, wv.T, precision=lax.Precision.HIGHEST,
                         preferred_element_type=jnp.float32)
    u = jnp.dot(att_ref[...], wv, precision=lax.Precision.HIGHEST,
                preferred_element_type=jnp.float32)
    asd_ref[...] = lax.dot_general(
        u, xv, (((1,), (1,)), ((), ())),
        precision=lax.Precision.HIGHEST,
        preferred_element_type=jnp.float32)


def _sc_body(h_hbm, edge_hbm, a_hbm, acc_hbm, den_hbm,
             as_v, ad_v, idx0, idx1, dst0, dst1, s2_0, s2_1,
             ex0, ex1, rows0, rows1, st0, st1, zbuf, zden,
             acc_sh, den_sh, sem_g0, sem_g1):
    cid = lax.axis_index("c")
    sid = lax.axis_index("s")

    pltpu.sync_copy(a_hbm.at[pl.ds(0, _N)], as_v)
    pltpu.sync_copy(a_hbm.at[pl.ds(_N, _N)], ad_v)

    z16 = jnp.zeros((16,), jnp.float32)
    for r in range(_ZR):
        zden[r, :] = z16
        for c in range(_DH // 16):
            zbuf[r, pl.ds(c * 16, 16)] = z16
    for r in range(_B):
        st0[r, :] = z16
        st1[r, :] = z16

    row0 = sid * _RPW

    @pl.loop(0, _RPW // _ZR)
    def _zero(k):
        pltpu.sync_copy(zbuf, acc_sh.at[pl.ds(row0 + k * _ZR, _ZR)])

        @pl.when(cid == 0)
        def _():
            pltpu.sync_copy(zden, den_sh.at[pl.ds(row0 + k * _ZR, _ZR)])
    del _zero

    plsc.subcore_barrier()

    iota16 = lax.iota(jnp.int32, 16)
    zi16 = jnp.zeros((16,), jnp.int32)
    base = sid * _NB

    def _prep(i, idx_v, dst_v, s2_v, ex_v, st_v, rows_v, sem):
        pltpu.sync_copy(edge_hbm.at[i], idx_v)
        for g in range(_G):
            sl = pl.ds(g * 16, 16)
            sv = idx_v[sl]
            dv = idx_v[pl.ds(_B + g * 16, 16)]
            s2_v[sl] = sv * 2 + cid
            dst_v[sl] = dv
        gat = pltpu.async_copy(h_hbm.at[s2_v], rows_v, sem)
        del gat
        for g in range(_G):
            sl = pl.ds(g * 16, 16)
            sg = idx_v[sl]
            dg = dst_v[sl]
            e = plsc.load_gather(as_v, [sg]) + plsc.load_gather(ad_v, [dg])
            e = jnp.where(e > 0.0, e, 0.2 * e)
            ex16 = jnp.exp(e)
            ex_v[pl.ds(16 + g * 16, 16)] = ex16
            plsc.store_scatter(st_v, [g * 16 + iota16, zi16], ex16)

    def _proc(dst_v, s2_v, ex_v, st_v, rows_v, sem):
        pltpu.make_async_copy(h_hbm.at[s2_v], rows_v, sem).wait()
        for e_ in range(_B):
            splat = plsc.load_gather(
                ex_v, [jnp.full((16,), 16 + e_, jnp.int32)])
            for c in range(_DH // 16):
                sl = pl.ds(c * 16, 16)
                rows_v[e_, sl] = rows_v[e_, sl] * splat
        pltpu.sync_copy(rows_v, acc_sh.at[dst_v], add=True)

        @pl.when(cid == 0)
        def _():
            pltpu.sync_copy(st_v, den_sh.at[dst_v], add=True)

    _prep(base, idx0, dst0, s2_0, ex0, st0, rows0, sem_g0)

    @pl.loop(0, _NB // 2)
    def _pair(k):
        i0 = base + 2 * k
        _prep(i0 + 1, idx1, dst1, s2_1, ex1, st1, rows1, sem_g1)
        _proc(dst0, s2_0, ex0, st0, rows0, sem_g0)

        @pl.when(k < _NB // 2 - 1)
        def _():
            _prep(i0 + 2, idx0, dst0, s2_0, ex0, st0, rows0, sem_g0)

        _proc(dst1, s2_1, ex1, st1, rows1, sem_g1)
    del _pair

    plsc.subcore_barrier()

    pltpu.sync_copy(acc_sh.at[pl.ds(row0, _RPW)],
                    acc_hbm.at[cid, pl.ds(row0, _RPW)])

    @pl.when(cid == 0)
    def _copy_den():
        pltpu.sync_copy(den_sh.at[pl.ds(row0, _RPW)],
                        den_hbm.at[pl.ds(row0, _RPW)])
    del _copy_den


def _post_body(acc_ref, den_ref, x_ref, bias_ref, gamma_ref, beta_ref, o_ref):
    out = jnp.concatenate([acc_ref[0, :_N], acc_ref[1, :_N]], axis=1)
    den = den_ref[:_N, 0]
    out = out / (den + 1e-16)[:, None] + bias_ref[...][None, :]
    mean = jnp.mean(out, axis=0)
    cent = out - mean[None, :]
    var = jnp.mean(cent * cent, axis=0)
    y = cent * lax.rsqrt(var + 1e-5) * gamma_ref[...][None, :] \
        + beta_ref[...][None, :]
    o_ref[...] = x_ref[...] + jnp.maximum(y, 0.0)


def kernel(x, edge_index, W, att_src, att_dst, bias, gamma, beta):
    h, asd = pl.pallas_call(
        _pre_body,
        out_shape=[
            jax.ShapeDtypeStruct((_N, _D), jnp.float32),
            jax.ShapeDtypeStruct((2, _N), jnp.float32),
        ],
    )(x, W, jnp.stack([att_src, att_dst]))

    edges = edge_index.reshape(2, _NBT, _B).transpose(1, 0, 2) \
                      .reshape(_NBT, 2 * _B)

    sc_fn = pl.kernel(
        _sc_body,
        out_type=[
            jax.ShapeDtypeStruct((_NC, _NP, _DH), jnp.float32),
            jax.ShapeDtypeStruct((_NP, 16), jnp.float32),
        ],
        mesh=plsc.VectorSubcoreMesh(core_axis_name="c", subcore_axis_name="s"),
        compiler_params=_sc_compiler_params(),
        scratch_types=[
            pltpu.VMEM((_N,), jnp.float32),
            pltpu.VMEM((_N,), jnp.float32),
            pltpu.VMEM((2 * _B,), jnp.int32),
            pltpu.VMEM((2 * _B,), jnp.int32),
            pltpu.VMEM((_B,), jnp.int32),
            pltpu.VMEM((_B,), jnp.int32),
            pltpu.VMEM((_B,), jnp.int32),
            pltpu.VMEM((_B,), jnp.int32),
            pltpu.VMEM((16 + _B,), jnp.float32),
            pltpu.VMEM((16 + _B,), jnp.float32),
            pltpu.VMEM((_B, _DH), jnp.float32),
            pltpu.VMEM((_B, _DH), jnp.float32),
            pltpu.VMEM((_B, 16), jnp.float32),
            pltpu.VMEM((_B, 16), jnp.float32),
            pltpu.VMEM((_ZR, _DH), jnp.float32),
            pltpu.VMEM((_ZR, 16), jnp.float32),
            pltpu.VMEM_SHARED((_NP, _DH), jnp.float32),
            pltpu.VMEM_SHARED((_NP, 16), jnp.float32),
            pltpu.SemaphoreType.DMA,
            pltpu.SemaphoreType.DMA,
        ],
    )
    acc, den = sc_fn(h.reshape(2 * _N, _DH), edges, asd.reshape(2 * _N))

    return pl.pallas_call(
        _post_body,
        out_shape=jax.ShapeDtypeStruct((_N, _D), jnp.float32),
    )(acc, den, x, bias, gamma, beta)
